# baseline (speedup 1.0000x reference)
import functools

import jax
import jax.numpy as jnp
from jax import lax
from jax.experimental import pallas as pl
from jax.experimental.pallas import tpu as pltpu
from jax.experimental.pallas import tpu_sc as plsc

N_NODES = 10000
D = 128
NP = 10240
NG = 64
NC = 2
NS = 16
K = 112
E = 320000
NBUF = 3
TCHUNKS = 180
CHUNKS0 = 66
E_PAD = NS * TCHUNKS * K
RPT = NP // NS


@functools.cache
def _mesh():
    return plsc.VectorSubcoreMesh(
        core_axis_name="c", subcore_axis_name="s", num_cores=NC, num_subcores=NS
    )


def _zero_rows(ref, nrows, ncols):
    zer = jnp.zeros((16,), jnp.float32)
    lanes = ncols // 16

    def body(i, _):
        r = i // lanes
        l = (i % lanes) * 16
        ref[r, pl.ds(l, 16)] = zer
        return 0

    lax.fori_loop(0, nrows * lanes, body, 0, unroll=False)


def _zero_acc_slice(acc, s, zb):
    for t in range(RPT // K):
        pltpu.sync_copy(zb, acc.at[pl.ds(s * RPT + t * K, K)])
    rrem = RPT - (RPT // K) * K
    if rrem:
        pltpu.sync_copy(zb.at[pl.ds(0, rrem)],
                        acc.at[pl.ds(s * RPT + (RPT // K) * K, rrem)])


def _deg_body(dst_hbm, out_hbm, ones_v, dv, acc, isems, ssems):
    c = lax.axis_index("c")
    s = lax.axis_index("s")
    base = (c * NS + s) * (TCHUNKS // 2) * K

    def start_idx(j, b):
        pltpu.async_copy(dst_hbm.at[pl.ds(base + j * K, K)], dv[b],
                         isems.at[b])

    def wait_idx(b):
        pltpu.make_async_copy(dst_hbm.at[pl.ds(0, K)], dv[b],
                              isems.at[b]).wait()

    def start_scatter(b):
        pltpu.async_copy(ones_v, acc.at[dv[b]], ssems.at[b], add=True)

    def wait_scatter(b):
        pltpu.make_async_copy(ones_v, acc.at[dv[b]], ssems.at[b]).wait()

    start_idx(0, 0)
    start_idx(1, 1)
    one = jnp.ones((16,), jnp.float32)

    def fill(i, _):
        ones_v[i, :] = one
        return 0

    lax.fori_loop(0, K, fill, 0, unroll=False)

    def zscope(zb):
        _zero_rows(zb, K, 16)
        _zero_acc_slice(acc, s, zb)

    pl.run_scoped(zscope, pltpu.VMEM((K, 16), jnp.float32))
    plsc.subcore_barrier()

    nchunks = TCHUNKS // 2
    main = (nchunks // NBUF) * NBUF
    jj_n = main // NBUF

    def body(jj, _):
        for b in range(NBUF):
            j = jj * NBUF + b
            wait_idx(b)
            start_scatter(b)
            if b == 0:
                @pl.when(jj > 0)
                def _():
                    wait_scatter((b + NBUF - 1) % NBUF)
            else:
                wait_scatter((b + NBUF - 1) % NBUF)
            t2 = (main - 3 - b) // NBUF
            if t2 >= jj_n - 1:
                start_idx(j + 2, (b + 2) % NBUF)
            else:
                @pl.when(jj <= t2)
                def _():
                    start_idx(j + 2, (b + 2) % NBUF)
        return 0

    lax.fori_loop(0, jj_n, body, 0, unroll=False)
    wait_scatter(NBUF - 1)
    for jx in range(main, nchunks):
        pltpu.sync_copy(dst_hbm.at[pl.ds(base + jx * K, K)], dv[0])
        pltpu.sync_copy(ones_v, acc.at[dv[0]], add=True)
    plsc.subcore_barrier()
    r0 = s * RPT
    pltpu.sync_copy(acc.at[pl.ds(r0, RPT)], out_hbm.at[c, pl.ds(r0, RPT)])


@functools.cache
def _deg_kernel():
    return pl.kernel(
        _deg_body,
        out_type=jax.ShapeDtypeStruct((NC, NP, 16), jnp.float32),
        mesh=_mesh(),
        scratch_types=[
            pltpu.VMEM((K, 16), jnp.float32),
            [pltpu.VMEM((K,), jnp.int32) for _ in range(NBUF)],
            pltpu.VMEM_SHARED((NP, 16), jnp.float32),
            pltpu.SemaphoreType.DMA((NBUF,)),
            pltpu.SemaphoreType.DMA((NBUF,)),
        ],
    )


def _agg_body(xs_hbm, src_hbm, dst_hbm, out_hbm, sv, dv, rv, acc,
              isems, gsems, ssems):
    c = lax.axis_index("c")
    s = lax.axis_index("s")
    chunks1 = TCHUNKS - CHUNKS0
    if CHUNKS0 == chunks1:
        base = (c * NS + s) * CHUNKS0 * K
    else:
        base = (c * (NS * CHUNKS0) + s * jnp.where(c == 0, CHUNKS0, chunks1)
                ) * K

    def start_idx(j, b):
        off = base + j * K
        pltpu.async_copy(src_hbm.at[pl.ds(off, K)], sv[b], isems.at[b])
        pltpu.async_copy(dst_hbm.at[pl.ds(off, K)], dv[b], isems.at[b])

    def wait_idx(b):
        pltpu.make_async_copy(src_hbm.at[pl.ds(0, K)], sv[b],
                              isems.at[b]).wait()
        pltpu.make_async_copy(dst_hbm.at[pl.ds(0, K)], dv[b],
                              isems.at[b]).wait()

    def start_gather(b):
        pltpu.async_copy(xs_hbm.at[sv[b]], rv[b], gsems.at[b])

    def wait_gather(b):
        pltpu.make_async_copy(xs_hbm.at[sv[b]], rv[b], gsems.at[b]).wait()

    def start_scatter(b):
        pltpu.async_copy(rv[b], acc.at[dv[b]], ssems.at[b], add=True)

    def wait_scatter(b):
        pltpu.make_async_copy(rv[b], acc.at[dv[b]], ssems.at[b]).wait()

    start_idx(0, 0)
    start_idx(1, 1)
    _zero_rows(rv[0], K, D)
    _zero_acc_slice(acc, s, rv[0])
    plsc.subcore_barrier()
    wait_idx(0)
    start_gather(0)

    def impl(nchunks):
        main = (nchunks // NBUF) * NBUF
        jj_n = main // NBUF

        def body(jj, _):
            for b in range(NBUF):
                j = jj * NBUF + b
                pb = (b + NBUF - 1) % NBUF
                nb = (b + 1) % NBUF
                if b == 0:
                    @pl.when(jj > 0)
                    def _():
                        wait_scatter(pb)
                else:
                    wait_scatter(pb)
                t2 = (main - 3 - b) // NBUF
                if t2 >= jj_n - 1:
                    start_idx(j + 2, (b + 2) % NBUF)
                else:
                    @pl.when(jj <= t2)
                    def _():
                        start_idx(j + 2, (b + 2) % NBUF)
                wait_gather(b)
                start_scatter(b)
                t1 = (main - 2 - b) // NBUF
                if t1 >= jj_n - 1:
                    wait_idx(nb)
                    start_gather(nb)
                else:
                    @pl.when(jj <= t1)
                    def _():
                        wait_idx(nb)
                        start_gather(nb)
            return 0

        lax.fori_loop(0, jj_n, body, 0, unroll=False)
        wait_scatter(NBUF - 1)
        for jx in range(main, nchunks):
            off = base + jx * K
            pltpu.sync_copy(src_hbm.at[pl.ds(off, K)], sv[0])
            pltpu.sync_copy(dst_hbm.at[pl.ds(off, K)], dv[0])
            pltpu.sync_copy(xs_hbm.at[sv[0]], rv[0])
            pltpu.sync_copy(rv[0], acc.at[dv[0]], add=True)

    if CHUNKS0 == chunks1:
        impl(CHUNKS0)
    else:
        @pl.when(c == 0)
        def _():
            impl(CHUNKS0)

        @pl.when(c == 1)
        def _():
            impl(chunks1)

    plsc.subcore_barrier()
    r0 = s * RPT
    pltpu.sync_copy(acc.at[pl.ds(r0, RPT)], out_hbm.at[c, pl.ds(r0, RPT)])


@functools.cache
def _agg_kernel():
    return pl.kernel(
        _agg_body,
        out_type=jax.ShapeDtypeStruct((NC, NP, D), jnp.float32),
        mesh=_mesh(),
        scratch_types=[
            [pltpu.VMEM((K,), jnp.int32) for _ in range(NBUF)],
            [pltpu.VMEM((K,), jnp.int32) for _ in range(NBUF)],
            [pltpu.VMEM((K, D), jnp.float32) for _ in range(NBUF)],
            pltpu.VMEM_SHARED((NP, D), jnp.float32),
            pltpu.SemaphoreType.DMA((NBUF,)),
            pltpu.SemaphoreType.DMA((NBUF,)),
            pltpu.SemaphoreType.DMA((NBUF,)),
        ],
    )


_BLK = 1280
_GRID = NP // _BLK


def _tc_b_body(x_ref, w1_ref, dp_ref, xw1_ref, xs1_ref, dis_ref):
    xw = jnp.dot(x_ref[...], w1_ref[...], preferred_element_type=jnp.float32)
    deg = dp_ref[0, :, 0:1] + dp_ref[1, :, 0:1]
    dis = lax.rsqrt(1.0 + deg)
    xw1_ref[...] = xw
    dis_ref[...] = dis
    xs1_ref[...] = xw * dis


def _tc_b(x_p, W1, deg_part):
    return pl.pallas_call(
        _tc_b_body,
        grid=(_GRID,),
        in_specs=[
            pl.BlockSpec((_BLK, D), lambda i: (i, 0)),
            pl.BlockSpec((D, D), lambda i: (0, 0)),
            pl.BlockSpec((NC, _BLK, 16), lambda i: (0, i, 0)),
        ],
        out_specs=[
            pl.BlockSpec((_BLK, D), lambda i: (i, 0)),
            pl.BlockSpec((_BLK, D), lambda i: (i, 0)),
            pl.BlockSpec((_BLK, 1), lambda i: (i, 0)),
        ],
        out_shape=[
            jax.ShapeDtypeStruct((NP, D), jnp.float32),
            jax.ShapeDtypeStruct((NP, D), jnp.float32),
            jax.ShapeDtypeStruct((NP, 1), jnp.float32),
        ],
    )(x_p, W1, deg_part)


def _tc_d_body(a_ref, xw1_ref, dis_ref, b1_ref, w2_ref, xw2_ref, xs2_ref):
    d = dis_ref[...]
    agg = a_ref[0] + a_ref[1]
    h = jnp.maximum(d * agg + (d * d) * xw1_ref[...] + b1_ref[...], 0.0)
    xw2 = jnp.dot(h, w2_ref[...], preferred_element_type=jnp.float32)
    xw2_ref[...] = xw2
    xs2_ref[...] = xw2 * d


def _tc_d(part1, xw1, dis, b1r, W2):
    return pl.pallas_call(
        _tc_d_body,
        grid=(_GRID,),
        in_specs=[
            pl.BlockSpec((NC, _BLK, D), lambda i: (0, i, 0)),
            pl.BlockSpec((_BLK, D), lambda i: (i, 0)),
            pl.BlockSpec((_BLK, 1), lambda i: (i, 0)),
            pl.BlockSpec((1, D), lambda i: (0, 0)),
            pl.BlockSpec((D, D), lambda i: (0, 0)),
        ],
        out_specs=[
            pl.BlockSpec((_BLK, D), lambda i: (i, 0)),
            pl.BlockSpec((_BLK, D), lambda i: (i, 0)),
        ],
        out_shape=[
            jax.ShapeDtypeStruct((NP, D), jnp.float32),
            jax.ShapeDtypeStruct((NP, D), jnp.float32),
        ],
    )(part1, xw1, dis, b1r, W2)


def _tc_f_body(a_ref, xw2_ref, dis_ref, b2_ref, bt_ref, wlp_ref, out_ref, acc):
    i = pl.program_id(0)

    @pl.when(i == 0)
    def _():
        acc[...] = jnp.zeros_like(acc)

    d = dis_ref[...]
    agg = a_ref[0] + a_ref[1]
    h2 = jnp.maximum(d * agg + (d * d) * xw2_ref[...] + b2_ref[...], 0.0)
    bt = bt_ref[0, 0, :]
    gids = lax.broadcasted_iota(jnp.int32, (NG, _BLK), 0)
    eqf = (gids == bt[None, :]).astype(jnp.float32)
    acc[...] += jnp.dot(eqf, h2, preferred_element_type=jnp.float32)

    @pl.when(i == _GRID - 1)
    def _():
        out_ref[...] = jnp.dot(acc[...], wlp_ref[...],
                               preferred_element_type=jnp.float32)


def _tc_f(part2, xw2, dis, b2r, bt3, Wlp):
    return pl.pallas_call(
        _tc_f_body,
        grid=(_GRID,),
        in_specs=[
            pl.BlockSpec((NC, _BLK, D), lambda i: (0, i, 0)),
            pl.BlockSpec((_BLK, D), lambda i: (i, 0)),
            pl.BlockSpec((_BLK, 1), lambda i: (i, 0)),
            pl.BlockSpec((1, D), lambda i: (0, 0)),
            pl.BlockSpec((1, 1, _BLK), lambda i: (i, 0, 0)),
            pl.BlockSpec((D, D), lambda i: (0, 0)),
        ],
        out_specs=pl.BlockSpec((NG, D), lambda i: (0, 0)),
        out_shape=jax.ShapeDtypeStruct((NG, D), jnp.float32),
        scratch_shapes=[pltpu.VMEM((NG, D), jnp.float32)],
    )(part2, xw2, dis, b2r, bt3, Wlp)


def kernel(x, edge_index, batch, W1, b1, W2, b2, Wl, bl):
    padi = jnp.full((E_PAD - E,), N_NODES, jnp.int32)
    src_p = jnp.concatenate([edge_index[0].astype(jnp.int32), padi])
    dst_p = jnp.concatenate([edge_index[1].astype(jnp.int32), padi])
    x_p = jnp.pad(x, ((0, NP - N_NODES), (0, 0)))
    bt3 = jnp.concatenate(
        [batch.astype(jnp.int32), jnp.full((NP - N_NODES,), NG, jnp.int32)]
    ).reshape(_GRID, 1, _BLK)
    b1r = b1.reshape(1, D)
    b2r = b2.reshape(1, D)
    Wlp = jnp.pad(Wl, ((0, 0), (0, D - Wl.shape[1])))

    deg_part = _deg_kernel()(dst_p)
    xw1, xs1, dis = _tc_b(x_p, W1, deg_part)
    part1 = _agg_kernel()(xs1, src_p, dst_p)
    xw2, xs2 = _tc_d(part1, xw1, dis, b1r, W2)
    part2 = _agg_kernel()(xs2, src_p, dst_p)
    outf = _tc_f(part2, xw2, dis, b2r, bt3, Wlp)
    return outf[:, :1] + bl


# speedup vs baseline: 1.1790x; 1.1790x over previous
import functools

import jax
import jax.numpy as jnp
from jax import lax
from jax.experimental import pallas as pl
from jax.experimental.pallas import tpu as pltpu
from jax.experimental.pallas import tpu_sc as plsc

N_NODES = 10000
D = 128
NP = 10240
NG = 64
NC = 2
NS = 16
K = 112
E = 320000
NBUF = 3
TCHUNKS = 180
CHUNKS0 = 114
E_PAD = NS * TCHUNKS * K
RPT = NP // NS


@functools.cache
def _mesh():
    return plsc.VectorSubcoreMesh(
        core_axis_name="c", subcore_axis_name="s", num_cores=NC, num_subcores=NS
    )


def _zero_rows(ref, nrows, ncols):
    zer = jnp.zeros((16,), jnp.float32)
    lanes = ncols // 16

    def body(i, _):
        r = i // lanes
        l = (i % lanes) * 16
        ref[r, pl.ds(l, 16)] = zer
        return 0

    lax.fori_loop(0, nrows * lanes, body, 0, unroll=False)


def _zero_acc_slice(acc, s, zb):
    for t in range(RPT // K):
        pltpu.sync_copy(zb, acc.at[pl.ds(s * RPT + t * K, K)])
    rrem = RPT - (RPT // K) * K
    if rrem:
        pltpu.sync_copy(zb.at[pl.ds(0, rrem)],
                        acc.at[pl.ds(s * RPT + (RPT // K) * K, rrem)])


def _deg_body(dst_hbm, out_hbm, ones_v, dv, acc, isems, ssems):
    c = lax.axis_index("c")
    s = lax.axis_index("s")
    base = (c * NS + s) * (TCHUNKS // 2) * K

    def start_idx(j, b):
        pltpu.async_copy(dst_hbm.at[pl.ds(base + j * K, K)], dv[b],
                         isems.at[b])

    def wait_idx(b):
        pltpu.make_async_copy(dst_hbm.at[pl.ds(0, K)], dv[b],
                              isems.at[b]).wait()

    def start_scatter(b):
        pltpu.async_copy(ones_v, acc.at[dv[b]], ssems.at[b], add=True)

    def wait_scatter(b):
        pltpu.make_async_copy(ones_v, acc.at[dv[b]], ssems.at[b]).wait()

    start_idx(0, 0)
    start_idx(1, 1)
    one = jnp.ones((16,), jnp.float32)

    def fill(i, _):
        ones_v[i, :] = one
        return 0

    lax.fori_loop(0, K, fill, 0, unroll=False)

    def zscope(zb):
        _zero_rows(zb, K, 16)
        _zero_acc_slice(acc, s, zb)

    pl.run_scoped(zscope, pltpu.VMEM((K, 16), jnp.float32))
    plsc.subcore_barrier()

    nchunks = TCHUNKS // 2
    main = (nchunks // NBUF) * NBUF
    jj_n = main // NBUF

    def body(jj, _):
        for b in range(NBUF):
            j = jj * NBUF + b
            wait_idx(b)
            start_scatter(b)
            if b == 0:
                @pl.when(jj > 0)
                def _():
                    wait_scatter((b + NBUF - 1) % NBUF)
            else:
                wait_scatter((b + NBUF - 1) % NBUF)
            t2 = (main - 3 - b) // NBUF
            if t2 >= jj_n - 1:
                start_idx(j + 2, (b + 2) % NBUF)
            else:
                @pl.when(jj <= t2)
                def _():
                    start_idx(j + 2, (b + 2) % NBUF)
        return 0

    lax.fori_loop(0, jj_n, body, 0, unroll=False)
    wait_scatter(NBUF - 1)
    for jx in range(main, nchunks):
        pltpu.sync_copy(dst_hbm.at[pl.ds(base + jx * K, K)], dv[0])
        pltpu.sync_copy(ones_v, acc.at[dv[0]], add=True)
    plsc.subcore_barrier()
    r0 = s * RPT
    pltpu.sync_copy(acc.at[pl.ds(r0, RPT)], out_hbm.at[c, pl.ds(r0, RPT)])


@functools.cache
def _deg_kernel():
    return pl.kernel(
        _deg_body,
        out_type=jax.ShapeDtypeStruct((NC, NP, 16), jnp.float32),
        mesh=_mesh(),
        scratch_types=[
            pltpu.VMEM((K, 16), jnp.float32),
            [pltpu.VMEM((K,), jnp.int32) for _ in range(NBUF)],
            pltpu.VMEM_SHARED((NP, 16), jnp.float32),
            pltpu.SemaphoreType.DMA((NBUF,)),
            pltpu.SemaphoreType.DMA((NBUF,)),
        ],
    )


def _agg_body(xs_hbm, src_hbm, dst_hbm, out_hbm, sv, dv, rv, acc,
              isems, gsems, ssems):
    c = lax.axis_index("c")
    s = lax.axis_index("s")
    chunks1 = TCHUNKS - CHUNKS0
    if CHUNKS0 == chunks1:
        base = (c * NS + s) * CHUNKS0 * K
    else:
        base = (c * (NS * CHUNKS0) + s * jnp.where(c == 0, CHUNKS0, chunks1)
                ) * K

    def start_idx(j, b):
        off = base + j * K
        pltpu.async_copy(src_hbm.at[pl.ds(off, K)], sv[b], isems.at[b])
        pltpu.async_copy(dst_hbm.at[pl.ds(off, K)], dv[b], isems.at[b])

    def wait_idx(b):
        pltpu.make_async_copy(src_hbm.at[pl.ds(0, K)], sv[b],
                              isems.at[b]).wait()
        pltpu.make_async_copy(dst_hbm.at[pl.ds(0, K)], dv[b],
                              isems.at[b]).wait()

    def start_gather(b):
        pltpu.async_copy(xs_hbm.at[sv[b]], rv[b], gsems.at[b])

    def wait_gather(b):
        pltpu.make_async_copy(xs_hbm.at[sv[b]], rv[b], gsems.at[b]).wait()

    def start_scatter(b):
        pltpu.async_copy(rv[b], acc.at[dv[b]], ssems.at[b], add=True)

    def wait_scatter(b):
        pltpu.make_async_copy(rv[b], acc.at[dv[b]], ssems.at[b]).wait()

    start_idx(0, 0)
    start_idx(1, 1)
    _zero_rows(rv[0], K, D)
    _zero_acc_slice(acc, s, rv[0])
    plsc.subcore_barrier()
    wait_idx(0)
    start_gather(0)

    def impl(nchunks):
        main = (nchunks // NBUF) * NBUF
        jj_n = main // NBUF

        def body(jj, _):
            for b in range(NBUF):
                j = jj * NBUF + b
                pb = (b + NBUF - 1) % NBUF
                nb = (b + 1) % NBUF
                if b == 0:
                    @pl.when(jj > 0)
                    def _():
                        wait_scatter(pb)
                else:
                    wait_scatter(pb)
                t2 = (main - 3 - b) // NBUF
                if t2 >= jj_n - 1:
                    start_idx(j + 2, (b + 2) % NBUF)
                else:
                    @pl.when(jj <= t2)
                    def _():
                        start_idx(j + 2, (b + 2) % NBUF)
                wait_gather(b)
                start_scatter(b)
                t1 = (main - 2 - b) // NBUF
                if t1 >= jj_n - 1:
                    wait_idx(nb)
                    start_gather(nb)
                else:
                    @pl.when(jj <= t1)
                    def _():
                        wait_idx(nb)
                        start_gather(nb)
            return 0

        lax.fori_loop(0, jj_n, body, 0, unroll=False)
        wait_scatter(NBUF - 1)
        for jx in range(main, nchunks):
            off = base + jx * K
            pltpu.sync_copy(src_hbm.at[pl.ds(off, K)], sv[0])
            pltpu.sync_copy(dst_hbm.at[pl.ds(off, K)], dv[0])
            pltpu.sync_copy(xs_hbm.at[sv[0]], rv[0])
            pltpu.sync_copy(rv[0], acc.at[dv[0]], add=True)

    if CHUNKS0 == chunks1:
        impl(CHUNKS0)
    else:
        @pl.when(c == 0)
        def _():
            impl(CHUNKS0)

        @pl.when(c == 1)
        def _():
            impl(chunks1)

    plsc.subcore_barrier()
    r0 = s * RPT
    pltpu.sync_copy(acc.at[pl.ds(r0, RPT)], out_hbm.at[c, pl.ds(r0, RPT)])


@functools.cache
def _agg_kernel():
    return pl.kernel(
        _agg_body,
        out_type=jax.ShapeDtypeStruct((NC, NP, D), jnp.float32),
        mesh=_mesh(),
        scratch_types=[
            [pltpu.VMEM((K,), jnp.int32) for _ in range(NBUF)],
            [pltpu.VMEM((K,), jnp.int32) for _ in range(NBUF)],
            [pltpu.VMEM((K, D), jnp.float32) for _ in range(NBUF)],
            pltpu.VMEM_SHARED((NP, D), jnp.float32),
            pltpu.SemaphoreType.DMA((NBUF,)),
            pltpu.SemaphoreType.DMA((NBUF,)),
            pltpu.SemaphoreType.DMA((NBUF,)),
        ],
    )


_BLK = 1280
_GRID = NP // _BLK


def _tc_b_body(x_ref, w1_ref, dp_ref, xw1_ref, xs1_ref, dis_ref):
    xw = jnp.dot(x_ref[...], w1_ref[...], preferred_element_type=jnp.float32)
    deg = dp_ref[0, :, 0:1] + dp_ref[1, :, 0:1]
    dis = lax.rsqrt(1.0 + deg)
    xw1_ref[...] = xw
    dis_ref[...] = dis
    xs1_ref[...] = xw * dis


def _tc_b(x_p, W1, deg_part):
    return pl.pallas_call(
        _tc_b_body,
        grid=(_GRID,),
        in_specs=[
            pl.BlockSpec((_BLK, D), lambda i: (i, 0)),
            pl.BlockSpec((D, D), lambda i: (0, 0)),
            pl.BlockSpec((NC, _BLK, 16), lambda i: (0, i, 0)),
        ],
        out_specs=[
            pl.BlockSpec((_BLK, D), lambda i: (i, 0)),
            pl.BlockSpec((_BLK, D), lambda i: (i, 0)),
            pl.BlockSpec((_BLK, 1), lambda i: (i, 0)),
        ],
        out_shape=[
            jax.ShapeDtypeStruct((NP, D), jnp.float32),
            jax.ShapeDtypeStruct((NP, D), jnp.float32),
            jax.ShapeDtypeStruct((NP, 1), jnp.float32),
        ],
    )(x_p, W1, deg_part)


def _tc_d_body(a_ref, xw1_ref, dis_ref, b1_ref, w2_ref, xw2_ref, xs2_ref):
    d = dis_ref[...]
    agg = a_ref[0] + a_ref[1]
    h = jnp.maximum(d * agg + (d * d) * xw1_ref[...] + b1_ref[...], 0.0)
    xw2 = jnp.dot(h, w2_ref[...], preferred_element_type=jnp.float32)
    xw2_ref[...] = xw2
    xs2_ref[...] = xw2 * d


def _tc_d(part1, xw1, dis, b1r, W2):
    return pl.pallas_call(
        _tc_d_body,
        grid=(_GRID,),
        in_specs=[
            pl.BlockSpec((NC, _BLK, D), lambda i: (0, i, 0)),
            pl.BlockSpec((_BLK, D), lambda i: (i, 0)),
            pl.BlockSpec((_BLK, 1), lambda i: (i, 0)),
            pl.BlockSpec((1, D), lambda i: (0, 0)),
            pl.BlockSpec((D, D), lambda i: (0, 0)),
        ],
        out_specs=[
            pl.BlockSpec((_BLK, D), lambda i: (i, 0)),
            pl.BlockSpec((_BLK, D), lambda i: (i, 0)),
        ],
        out_shape=[
            jax.ShapeDtypeStruct((NP, D), jnp.float32),
            jax.ShapeDtypeStruct((NP, D), jnp.float32),
        ],
    )(part1, xw1, dis, b1r, W2)


def _tc_f_body(a_ref, xw2_ref, dis_ref, b2_ref, bt_ref, wlp_ref, out_ref, acc):
    i = pl.program_id(0)

    @pl.when(i == 0)
    def _():
        acc[...] = jnp.zeros_like(acc)

    d = dis_ref[...]
    agg = a_ref[0] + a_ref[1]
    h2 = jnp.maximum(d * agg + (d * d) * xw2_ref[...] + b2_ref[...], 0.0)
    bt = bt_ref[0, 0, :]
    gids = lax.broadcasted_iota(jnp.int32, (NG, _BLK), 0)
    eqf = (gids == bt[None, :]).astype(jnp.float32)
    acc[...] += jnp.dot(eqf, h2, preferred_element_type=jnp.float32)

    @pl.when(i == _GRID - 1)
    def _():
        out_ref[...] = jnp.dot(acc[...], wlp_ref[...],
                               preferred_element_type=jnp.float32)


def _tc_f(part2, xw2, dis, b2r, bt3, Wlp):
    return pl.pallas_call(
        _tc_f_body,
        grid=(_GRID,),
        in_specs=[
            pl.BlockSpec((NC, _BLK, D), lambda i: (0, i, 0)),
            pl.BlockSpec((_BLK, D), lambda i: (i, 0)),
            pl.BlockSpec((_BLK, 1), lambda i: (i, 0)),
            pl.BlockSpec((1, D), lambda i: (0, 0)),
            pl.BlockSpec((1, 1, _BLK), lambda i: (i, 0, 0)),
            pl.BlockSpec((D, D), lambda i: (0, 0)),
        ],
        out_specs=pl.BlockSpec((NG, D), lambda i: (0, 0)),
        out_shape=jax.ShapeDtypeStruct((NG, D), jnp.float32),
        scratch_shapes=[pltpu.VMEM((NG, D), jnp.float32)],
    )(part2, xw2, dis, b2r, bt3, Wlp)


def kernel(x, edge_index, batch, W1, b1, W2, b2, Wl, bl):
    padi = jnp.full((E_PAD - E,), N_NODES, jnp.int32)
    src_p = jnp.concatenate([edge_index[0].astype(jnp.int32), padi])
    dst_p = jnp.concatenate([edge_index[1].astype(jnp.int32), padi])
    x_p = jnp.pad(x, ((0, NP - N_NODES), (0, 0)))
    bt3 = jnp.concatenate(
        [batch.astype(jnp.int32), jnp.full((NP - N_NODES,), NG, jnp.int32)]
    ).reshape(_GRID, 1, _BLK)
    b1r = b1.reshape(1, D)
    b2r = b2.reshape(1, D)
    Wlp = jnp.pad(Wl, ((0, 0), (0, D - Wl.shape[1])))

    deg_part = _deg_kernel()(dst_p)
    xw1, xs1, dis = _tc_b(x_p, W1, deg_part)
    part1 = _agg_kernel()(xs1, src_p, dst_p)
    xw2, xs2 = _tc_d(part1, xw1, dis, b1r, W2)
    part2 = _agg_kernel()(xs2, src_p, dst_p)
    outf = _tc_f(part2, xw2, dis, b2r, bt3, Wlp)
    return outf[:, :1] + bl
